# baseline (speedup 1.0000x reference)
import functools

import jax
import jax.numpy as jnp
from jax import lax
from jax.experimental import pallas as pl
from jax.experimental.pallas import tpu as pltpu

L = 3
A = 16384
T = L * A
B = 8
G = 64
TOPK = 27
NQ = B * G
CPAD = 32
R = 128
C = 128


def _xor_shuffle(x, dist, axis, coord):
    n = x.shape[axis]
    m = pltpu.roll(x, n - dist, axis=axis)
    p = pltpu.roll(x, dist, axis=axis)
    bit = (coord & dist) != 0
    return jnp.where(bit, p, m)


def _sort_body(x0_ref, x1_ref, scx_ref, sw_ref, sidx_ref):
    x0 = x0_ref[...]
    x1 = x1_ref[...]
    key = 0.5 * (x0 + x1)
    sw_ref[...] = x1 - x0
    r = lax.broadcasted_iota(jnp.int32, (L, R, C), 1)
    c = lax.broadcasted_iota(jnp.int32, (L, R, C), 2)
    ii = r * C + c
    idx = ii
    for p in range(1, 15):
        asc = (ii & (1 << p)) == 0
        for q in range(p - 1, -1, -1):
            d = 1 << q
            if q < 7:
                axis, coord, dist = 2, c, d
            else:
                axis, coord, dist = 1, r, d >> 7
            key_p = _xor_shuffle(key, dist, axis, coord)
            idx_p = _xor_shuffle(idx, dist, axis, coord)
            low = (coord & dist) == 0
            want_min = low == asc
            new_key = jnp.where(want_min, jnp.minimum(key, key_p),
                                jnp.maximum(key, key_p))
            idx = jnp.where(new_key == key, idx, idx_p)
            key = new_key
    scx_ref[...] = key
    sidx_ref[...] = idx


def _sort_levels(x0, x1, interpret=False):
    x0r = x0.reshape(L, R, C)
    x1r = x1.reshape(L, R, C)
    out = pl.pallas_call(
        _sort_body,
        out_shape=[jax.ShapeDtypeStruct((L, R, C), jnp.float32),
                   jax.ShapeDtypeStruct((L, R, C), jnp.float32),
                   jax.ShapeDtypeStruct((L, R, C), jnp.int32)],
        interpret=interpret,
    )(x0r, x1r)
    scx, sw, sidx = out
    return scx.reshape(L, A), sw.reshape(L, A), sidx.reshape(L, A)


def _finish_body(counts_ref, a2g_ref, ann_ref, anc_ref, reg_ref, out_ref):
    cnt = jnp.sum(counts_ref[...].astype(jnp.float32), axis=1)
    nwin = jnp.sum(cnt, axis=1, keepdims=True)
    gidx = lax.broadcasted_iota(jnp.int32, (B, G), 1).astype(jnp.float32)
    count_full = cnt + jnp.where(gidx == 0.0, float(T) - nwin, 0.0)
    q = a2g_ref[...]
    jj = lax.broadcasted_iota(jnp.int32, (B, G, G), 2)
    oh = (q[:, :, None] == jj).astype(jnp.float32)
    ann = ann_ref[...]
    ab0 = jnp.sum(oh * ann[:, None, :, 0], axis=2)
    ab1 = jnp.sum(oh * ann[:, None, :, 1], axis=2)
    gw = ab1 - ab0
    gcx = ab0 + 0.5 * gw
    gw = jnp.maximum(gw, 1.0)
    anc = anc_ref[...]
    aw = (anc[:, 1] - anc[:, 0])[None, :]
    acx = (anc[:, 0] + 0.5 * (anc[:, 1] - anc[:, 0]))[None, :]
    tdx = ((gcx - acx) / aw) / jnp.float32(0.1)
    tdw = jnp.log(gw / aw) / jnp.float32(0.2)
    reg = reg_ref[...]
    d0 = jnp.abs(tdx - reg[:, :, 0])
    d1 = jnp.abs(tdw - reg[:, :, 1])
    na = jnp.float32(3.0)

    def rl(d):
        return jnp.where(d <= 1.0 / na, 0.5 * na * d * d, d - 0.5 / na)

    f = rl(d0) + rl(d1)
    loss = jnp.sum(count_full * f, axis=1) / jnp.float32(2 * T)
    gate = jnp.sum(count_full * gidx, axis=1) > 0.0
    loss = jnp.where(gate, loss, 0.0)
    out_ref[...] = jnp.broadcast_to(loss[:, None], (B, 128))


def _finish(counts, a2g64, ann, anc64, reg64, interpret=False):
    out = pl.pallas_call(
        _finish_body,
        out_shape=jax.ShapeDtypeStruct((B, 128), jnp.float32),
        interpret=interpret,
    )(counts, a2g64, ann, anc64, reg64)
    return out[:, 0]


from jax.experimental.pallas import tpu_sc as plsc

NPOS = A - TOPK


def _sc_query(scx, sw, sidx, qb):
    mesh = plsc.VectorSubcoreMesh(core_axis_name="c", subcore_axis_name="s")

    @functools.partial(
        pl.kernel, mesh=mesh,
        compiler_params=pltpu.CompilerParams(needs_layout_passes=False),
        out_type=[jax.ShapeDtypeStruct((L, NQ, CPAD), jnp.float32),
                  jax.ShapeDtypeStruct((L, NQ, CPAD), jnp.int32),
                  jax.ShapeDtypeStruct((L, NQ, CPAD), jnp.int32)],
        scratch_types=[pltpu.VMEM((A,), jnp.float32),
                       pltpu.VMEM((A,), jnp.float32),
                       pltpu.VMEM((A,), jnp.int32),
                       pltpu.VMEM((4, NQ), jnp.float32),
                       pltpu.VMEM((G,), jnp.int32),
                       pltpu.VMEM((G, CPAD), jnp.float32),
                       pltpu.VMEM((G, CPAD), jnp.int32),
                       pltpu.VMEM((G, CPAD), jnp.int32)],
    )
    def qk(scx_hbm, sw_hbm, sidx_hbm, qb_hbm, ciou_hbm, ct_hbm, cin_hbm,
           scx_v, sw_v, sidx_v, qb_v, ss_v, oi_v, ot_v, oin_v):
        wid = lax.axis_index("s") * 2 + lax.axis_index("c")

        @pl.when(wid < 24)
        def _():
            lvl = wid // 8
            chunk = wid % 8
            qbase = chunk * G
            pltpu.sync_copy(scx_hbm.at[lvl], scx_v)
            pltpu.sync_copy(sw_hbm.at[lvl], sw_v)
            pltpu.sync_copy(sidx_hbm.at[lvl], sidx_v)
            pltpu.sync_copy(qb_hbm, qb_v)
            iota16 = lax.broadcasted_iota(jnp.int32, (16,), 0)
            for k in range(4):
                q2 = qb_v[2, pl.ds(qbase + k * 16, 16)]
                lo = jnp.zeros((16,), jnp.int32)
                hi = jnp.full((16,), NPOS, jnp.int32)

                def bs(_, carry):
                    lo, hi = carry
                    mid = (lo + hi) // 2
                    mid_c = jnp.minimum(mid, NPOS - 1)
                    va = plsc.load_gather(scx_v, [mid_c])
                    vb = plsc.load_gather(scx_v, [mid_c + TOPK])
                    upd = lo < hi
                    pred = (va + vb) < q2
                    lo = jnp.where(upd & pred, mid + 1, lo)
                    hi = jnp.where(upd & (~pred), mid, hi)
                    return lo, hi

                lo, hi = lax.fori_loop(0, 15, bs, (lo, hi))
                ss_v[pl.ds(k * 16, 16)] = lo

            for k in range(4):
                q16 = k * 16 + iota16
                svec = ss_v[pl.ds(k * 16, 16)]
                g0 = qb_v[0, pl.ds(qbase + k * 16, 16)]
                g1 = qb_v[1, pl.ds(qbase + k * 16, 16)]
                gw = g1 - g0

                def lane_body(lane, carry):
                    pos = svec + lane
                    ccx = plsc.load_gather(scx_v, [pos])
                    ctv = plsc.load_gather(sidx_v, [pos])
                    cw = plsc.load_gather(sw_v, [ctv])
                    cx0 = ccx - 0.5 * cw
                    cx1 = ccx + 0.5 * cw
                    iw = jnp.minimum(cx1, g1) - jnp.maximum(cx0, g0)
                    iw = jnp.maximum(iw, 0.0)
                    ua = jnp.maximum(cw + gw - iw, 1e-8)
                    iou = iw / ua
                    inside = jnp.minimum(ccx - g0, g1 - ccx) > 0.01
                    lsp = lane + jnp.zeros((16,), jnp.int32)
                    plsc.store_scatter(oi_v, [q16, lsp], iou)
                    plsc.store_scatter(ot_v, [q16, lsp], ctv)
                    plsc.store_scatter(oin_v, [q16, lsp],
                                       jnp.where(inside, 1, 0))
                    return carry

                lax.fori_loop(0, TOPK, lane_body, 0)
            pltpu.sync_copy(oi_v, ciou_hbm.at[lvl, pl.ds(qbase, G)])
            pltpu.sync_copy(ot_v, ct_hbm.at[lvl, pl.ds(qbase, G)])
            pltpu.sync_copy(oin_v, cin_hbm.at[lvl, pl.ds(qbase, G)])

    return qk(scx, sw, sidx, qb)


MING_INIT = 1 << 20


def _sc_scatter(ciou, ct, cin):
    mesh = plsc.VectorSubcoreMesh(core_axis_name="c", subcore_axis_name="s")

    @functools.partial(
        pl.kernel, mesh=mesh,
        compiler_params=pltpu.CompilerParams(needs_layout_passes=False),
        out_type=[jax.ShapeDtypeStruct((B, L, G), jnp.int32),
                  jax.ShapeDtypeStruct((B, G), jnp.int32)],
        scratch_types=[pltpu.VMEM((L, G, CPAD), jnp.float32),
                       pltpu.VMEM((G, CPAD), jnp.int32),
                       pltpu.VMEM((G, CPAD), jnp.int32),
                       pltpu.VMEM((G,), jnp.float32),
                       pltpu.VMEM((G + 16,), jnp.float32),
                       pltpu.VMEM((G + 16,), jnp.float32),
                       pltpu.VMEM((A,), jnp.float32),
                       pltpu.VMEM((A,), jnp.int32),
                       pltpu.VMEM((G,), jnp.int32),
                       pltpu.VMEM((G,), jnp.int32),
                       pltpu.VMEM((G, CPAD), jnp.int32)],
    )
    def sk(ciou_hbm, ct_hbm, cin_hbm, counts_hbm, a2g_hbm,
           civ, ctv, cinv, s1_v, mean_v, thr_v, maxiou_v, ming_v,
           cnt_v, a2g_v, ptl):
        wid = lax.axis_index("s") * 2 + lax.axis_index("c")

        @pl.when(wid < 24)
        def _():
            lvl = wid // 8
            b = wid % 8
            qbase = b * G
            for ll in range(L):
                pltpu.sync_copy(ciou_hbm.at[ll, pl.ds(qbase, G)], civ.at[ll])
            pltpu.sync_copy(ct_hbm.at[lvl, pl.ds(qbase, G)], ctv)
            pltpu.sync_copy(cin_hbm.at[lvl, pl.ds(qbase, G)], cinv)
            iota16 = lax.broadcasted_iota(jnp.int32, (16,), 0)
            masks = [(cc * 16 + iota16) < TOPK for cc in range(2)]
            lane0 = iota16 == 0

            def splat_i(s):
                return s + jnp.zeros((16,), jnp.int32)

            def sstore(ref, g, val):
                vec = val + jnp.zeros((16,), val.dtype)
                plsc.store_scatter(ref, [splat_i(g)], vec, mask=lane0)

            def sum_body(g, carry):
                acc = jnp.zeros((16,), jnp.float32)
                for ll in range(L):
                    for cc in range(2):
                        v = civ[ll, g, pl.ds(cc * 16, 16)]
                        acc = acc + jnp.where(masks[cc], v, 0.0)
                sstore(s1_v, g, jnp.sum(acc))
                return carry

            lax.fori_loop(0, G, sum_body, 0)
            for k in range(4):
                s1 = s1_v[pl.ds(k * 16, 16)]
                mean_v[pl.ds(k * 16, 16)] = s1 / 81.0

            def dev_body(g, carry):
                m = mean_v[pl.ds(g, 16)][0]
                acc = jnp.zeros((16,), jnp.float32)
                for ll in range(L):
                    for cc in range(2):
                        v = civ[ll, g, pl.ds(cc * 16, 16)] - m
                        v = jnp.where(masks[cc], v, 0.0)
                        acc = acc + v * v
                sstore(s1_v, g, jnp.sum(acc))
                return carry

            lax.fori_loop(0, G, dev_body, 0)
            for k in range(4):
                var = s1_v[pl.ds(k * 16, 16)] / 80.0
                vv = jnp.maximum(var, 1e-30)
                bi = plsc.bitcast(vv, jnp.int32)
                y = plsc.bitcast(0x5F3759DF - (bi >> 1), jnp.float32)
                for _ in range(4):
                    y = y * (1.5 - 0.5 * vv * y * y)
                std = vv * y
                thr_v[pl.ds(k * 16, 16)] = mean_v[pl.ds(k * 16, 16)] + std

            zf16 = jnp.zeros((16,), jnp.float32)
            mg16 = jnp.full((16,), MING_INIT, jnp.int32)
            for k in range(G // 16):
                maxiou_v[pl.ds(k * 16, 16)] = zf16
                ming_v[pl.ds(k * 16, 16)] = mg16

            def init_body(g, carry):
                for cc in range(2):
                    tl = ctv[g, pl.ds(cc * 16, 16)]
                    tl_c = jnp.where(masks[cc], tl, 0)
                    plsc.store_scatter(maxiou_v, [tl_c], zf16,
                                       mask=masks[cc])
                    plsc.store_scatter(ming_v, [tl_c], mg16,
                                       mask=masks[cc])
                return carry

            lax.fori_loop(0, G, init_body, 0)

            def pair_vecs(g, cc):
                iou = civ[lvl, g, pl.ds(cc * 16, 16)]
                tl = ctv[g, pl.ds(cc * 16, 16)]
                ins = cinv[g, pl.ds(cc * 16, 16)]
                thr = thr_v[pl.ds(g, 16)][0]
                pos = masks[cc] & (ins != 0) & (iou >= thr)
                tl_c = jnp.where(pos, tl, 0)
                return iou, tl_c, pos

            def smax_body(g, carry):
                for cc in range(2):
                    iou, tl_c, pos = pair_vecs(g, cc)
                    ptl[g, pl.ds(cc * 16, 16)] = jnp.where(pos, tl_c, -1)
                    cur = plsc.load_gather(maxiou_v, [tl_c])
                    upd = pos & (iou > cur)
                    plsc.store_scatter(maxiou_v, [tl_c], iou, mask=upd)
                return carry

            lax.fori_loop(0, G, smax_body, 0)

            def pair_vecs2(g, cc):
                iou = civ[lvl, g, pl.ds(cc * 16, 16)]
                tl = ptl[g, pl.ds(cc * 16, 16)]
                pos = tl >= 0
                tl_c = jnp.where(pos, tl, 0)
                return iou, tl_c, pos

            def smin_body(g, carry):
                gv = splat_i(g)
                for cc in range(2):
                    iou, tl_c, pos = pair_vecs2(g, cc)
                    mx = plsc.load_gather(maxiou_v, [tl_c])
                    el = pos & (iou == mx)
                    cur = plsc.load_gather(ming_v, [tl_c])
                    upd = el & (gv < cur)
                    plsc.store_scatter(ming_v, [tl_c], gv, mask=upd)
                return carry

            lax.fori_loop(0, G, smin_body, 0)

            def cnt_body(g, carry):
                gv = splat_i(g)
                tot = jnp.int32(0)
                for cc in range(2):
                    iou, tl_c, pos = pair_vecs2(g, cc)
                    mx = plsc.load_gather(maxiou_v, [tl_c])
                    mg = plsc.load_gather(ming_v, [tl_c])
                    win = pos & (iou == mx) & (gv == mg)
                    tot = tot + jnp.sum(jnp.where(win, 1, 0))
                sstore(cnt_v, g, tot)
                return carry

            lax.fori_loop(0, G, cnt_body, 0)
            pltpu.sync_copy(cnt_v, counts_hbm.at[b, lvl])

            @pl.when(lvl == 0)
            def _():
                for k in range(4):
                    idx = iota16 + k * 16
                    mi = plsc.load_gather(maxiou_v, [idx])
                    mg = plsc.load_gather(ming_v, [idx])
                    a2g_v[pl.ds(k * 16, 16)] = jnp.where(
                        mi > 0.0, mg, 0)
                pltpu.sync_copy(a2g_v, a2g_hbm.at[b])

    return sk(ciou, ct, cin)


def kernel(regressions, anchors, annotations):
    x0 = anchors[:, 0, :, 0]
    x1 = anchors[:, 0, :, 1]
    scx, sw, sidx = _sort_levels(x0, x1)
    g0 = annotations[:, :, 0].reshape(NQ)
    g1 = annotations[:, :, 1].reshape(NQ)
    qb = jnp.stack([g0, g1, g0 + g1, jnp.zeros_like(g0)], axis=0)
    ciou, ct, cin = _sc_query(scx, sw, sidx, qb)
    counts, a2g64 = _sc_scatter(ciou, ct, cin)
    return _finish(counts, a2g64, annotations, anchors[0, 0, :G, :],
                   regressions[:, :G, :])


# speedup vs baseline: 1.0185x; 1.0185x over previous
import functools

import jax
import jax.numpy as jnp
from jax import lax
from jax.experimental import pallas as pl
from jax.experimental.pallas import tpu as pltpu

L = 3
A = 16384
T = L * A
B = 8
G = 64
TOPK = 27
NQ = B * G
CPAD = 32
R = 128
C = 128


def _xor_shuffle(x, dist, axis, coord):
    n = x.shape[axis]
    m = pltpu.roll(x, n - dist, axis=axis)
    p = pltpu.roll(x, dist, axis=axis)
    bit = (coord & dist) != 0
    return jnp.where(bit, p, m)


def _sort_body(x0_ref, x1_ref, scx_ref, sw_ref, sidx_ref):
    x0 = x0_ref[...]
    x1 = x1_ref[...]
    key = 0.5 * (x0 + x1)
    sw_ref[...] = x1 - x0
    r = lax.broadcasted_iota(jnp.int32, (L, R, C), 1)
    c = lax.broadcasted_iota(jnp.int32, (L, R, C), 2)
    ii = r * C + c
    idx = ii
    for p in range(1, 15):
        asc = (ii & (1 << p)) == 0
        for q in range(p - 1, -1, -1):
            d = 1 << q
            if q < 7:
                axis, coord, dist = 2, c, d
            else:
                axis, coord, dist = 1, r, d >> 7
            key_p = _xor_shuffle(key, dist, axis, coord)
            idx_p = _xor_shuffle(idx, dist, axis, coord)
            low = (coord & dist) == 0
            want_min = low == asc
            new_key = jnp.where(want_min, jnp.minimum(key, key_p),
                                jnp.maximum(key, key_p))
            idx = jnp.where(new_key == key, idx, idx_p)
            key = new_key
    scx_ref[...] = key
    sidx_ref[...] = idx


def _sort_levels(x0, x1, interpret=False):
    x0r = x0.reshape(L, R, C)
    x1r = x1.reshape(L, R, C)
    out = pl.pallas_call(
        _sort_body,
        out_shape=[jax.ShapeDtypeStruct((L, R, C), jnp.float32),
                   jax.ShapeDtypeStruct((L, R, C), jnp.float32),
                   jax.ShapeDtypeStruct((L, R, C), jnp.int32)],
        interpret=interpret,
    )(x0r, x1r)
    scx, sw, sidx = out
    return scx.reshape(L, A), sw.reshape(L, A), sidx.reshape(L, A)


def _finish_body(counts_ref, a2g_ref, ann_ref, anc_ref, reg_ref, out_ref):
    cnt = jnp.sum(counts_ref[...].astype(jnp.float32), axis=1)
    nwin = jnp.sum(cnt, axis=1, keepdims=True)
    gidx = lax.broadcasted_iota(jnp.int32, (B, G), 1).astype(jnp.float32)
    count_full = cnt + jnp.where(gidx == 0.0, float(T) - nwin, 0.0)
    q = a2g_ref[...]
    jj = lax.broadcasted_iota(jnp.int32, (B, G, G), 2)
    oh = (q[:, :, None] == jj).astype(jnp.float32)
    ann = ann_ref[...]
    ab0 = jnp.sum(oh * ann[:, None, :, 0], axis=2)
    ab1 = jnp.sum(oh * ann[:, None, :, 1], axis=2)
    gw = ab1 - ab0
    gcx = ab0 + 0.5 * gw
    gw = jnp.maximum(gw, 1.0)
    anc = anc_ref[...]
    aw = (anc[:, 1] - anc[:, 0])[None, :]
    acx = (anc[:, 0] + 0.5 * (anc[:, 1] - anc[:, 0]))[None, :]
    tdx = ((gcx - acx) / aw) / jnp.float32(0.1)
    tdw = jnp.log(gw / aw) / jnp.float32(0.2)
    reg = reg_ref[...]
    d0 = jnp.abs(tdx - reg[:, :, 0])
    d1 = jnp.abs(tdw - reg[:, :, 1])
    na = jnp.float32(3.0)

    def rl(d):
        return jnp.where(d <= 1.0 / na, 0.5 * na * d * d, d - 0.5 / na)

    f = rl(d0) + rl(d1)
    loss = jnp.sum(count_full * f, axis=1) / jnp.float32(2 * T)
    gate = jnp.sum(count_full * gidx, axis=1) > 0.0
    loss = jnp.where(gate, loss, 0.0)
    out_ref[...] = jnp.broadcast_to(loss[:, None], (B, 128))


def _finish(counts, a2g64, ann, anc64, reg64, interpret=False):
    out = pl.pallas_call(
        _finish_body,
        out_shape=jax.ShapeDtypeStruct((B, 128), jnp.float32),
        interpret=interpret,
    )(counts, a2g64, ann, anc64, reg64)
    return out[:, 0]


from jax.experimental.pallas import tpu_sc as plsc

NPOS = A - TOPK


def _sc_query(scx, sw, sidx, qb):
    mesh = plsc.VectorSubcoreMesh(core_axis_name="c", subcore_axis_name="s")

    @functools.partial(
        pl.kernel, mesh=mesh,
        compiler_params=pltpu.CompilerParams(needs_layout_passes=False),
        out_type=[jax.ShapeDtypeStruct((L, NQ, CPAD), jnp.float32),
                  jax.ShapeDtypeStruct((L, NQ, CPAD), jnp.int32),
                  jax.ShapeDtypeStruct((L, NQ, CPAD), jnp.int32)],
        scratch_types=[pltpu.VMEM((A,), jnp.float32),
                       pltpu.VMEM((A,), jnp.float32),
                       pltpu.VMEM((A,), jnp.int32),
                       pltpu.VMEM((4, NQ), jnp.float32),
                       pltpu.VMEM((G,), jnp.int32),
                       pltpu.VMEM((G, CPAD), jnp.float32),
                       pltpu.VMEM((G, CPAD), jnp.int32),
                       pltpu.VMEM((G, CPAD), jnp.int32)],
    )
    def qk(scx_hbm, sw_hbm, sidx_hbm, qb_hbm, ciou_hbm, ct_hbm, cin_hbm,
           scx_v, sw_v, sidx_v, qb_v, ss_v, oi_v, ot_v, oin_v):
        wid = lax.axis_index("s") * 2 + lax.axis_index("c")

        @pl.when(wid < 24)
        def _():
            lvl = wid // 8
            chunk = wid % 8
            qbase = chunk * G
            pltpu.sync_copy(scx_hbm.at[lvl], scx_v)
            pltpu.sync_copy(sw_hbm.at[lvl], sw_v)
            pltpu.sync_copy(sidx_hbm.at[lvl], sidx_v)
            pltpu.sync_copy(qb_hbm, qb_v)
            iota16 = lax.broadcasted_iota(jnp.int32, (16,), 0)
            for k in range(4):
                q2 = qb_v[2, pl.ds(qbase + k * 16, 16)]
                lo = jnp.zeros((16,), jnp.int32)
                hi = jnp.full((16,), NPOS, jnp.int32)

                def bs(_, carry):
                    lo, hi = carry
                    mid = (lo + hi) // 2
                    mid_c = jnp.minimum(mid, NPOS - 1)
                    va = plsc.load_gather(scx_v, [mid_c])
                    vb = plsc.load_gather(scx_v, [mid_c + TOPK])
                    upd = lo < hi
                    pred = (va + vb) < q2
                    lo = jnp.where(upd & pred, mid + 1, lo)
                    hi = jnp.where(upd & (~pred), mid, hi)
                    return lo, hi

                lo, hi = lax.fori_loop(0, 15, bs, (lo, hi))
                ss_v[pl.ds(k * 16, 16)] = lo

            zeros16 = jnp.zeros((16,), jnp.int32)

            def cand_body(q, carry):
                qsplat = q + zeros16
                svec = plsc.load_gather(ss_v, [qsplat])
                g0 = plsc.load_gather(qb_v, [zeros16, qbase + qsplat])
                g1 = plsc.load_gather(qb_v, [zeros16 + 1, qbase + qsplat])
                for cc in range(2):
                    lane = cc * 16 + iota16
                    msk = lane < TOPK
                    pos = jnp.where(msk, svec + lane, 0)
                    ccx = plsc.load_gather(scx_v, [pos])
                    ctv = plsc.load_gather(sidx_v, [pos])
                    cw = plsc.load_gather(sw_v, [ctv])
                    cx0 = ccx - 0.5 * cw
                    cx1 = ccx + 0.5 * cw
                    iw = jnp.minimum(cx1, g1) - jnp.maximum(cx0, g0)
                    iw = jnp.maximum(iw, 0.0)
                    ua = jnp.maximum(cw + (g1 - g0) - iw, 1e-8)
                    iou = iw / ua
                    inside = jnp.minimum(ccx - g0, g1 - ccx) > 0.01
                    oi_v[q, pl.ds(cc * 16, 16)] = jnp.where(msk, iou, 0.0)
                    ot_v[q, pl.ds(cc * 16, 16)] = jnp.where(msk, ctv, 0)
                    oin_v[q, pl.ds(cc * 16, 16)] = jnp.where(msk & inside, 1, 0)
                return carry

            lax.fori_loop(0, G, cand_body, 0)
            pltpu.sync_copy(oi_v, ciou_hbm.at[lvl, pl.ds(qbase, G)])
            pltpu.sync_copy(ot_v, ct_hbm.at[lvl, pl.ds(qbase, G)])
            pltpu.sync_copy(oin_v, cin_hbm.at[lvl, pl.ds(qbase, G)])

    return qk(scx, sw, sidx, qb)


MING_INIT = 1 << 20


def _sc_scatter(ciou, ct, cin):
    mesh = plsc.VectorSubcoreMesh(core_axis_name="c", subcore_axis_name="s")

    @functools.partial(
        pl.kernel, mesh=mesh,
        compiler_params=pltpu.CompilerParams(needs_layout_passes=False),
        out_type=[jax.ShapeDtypeStruct((B, L, G), jnp.int32),
                  jax.ShapeDtypeStruct((B, G), jnp.int32)],
        scratch_types=[pltpu.VMEM((L, G, CPAD), jnp.float32),
                       pltpu.VMEM((G, CPAD), jnp.int32),
                       pltpu.VMEM((G, CPAD), jnp.int32),
                       pltpu.VMEM((G,), jnp.float32),
                       pltpu.VMEM((G + 16,), jnp.float32),
                       pltpu.VMEM((G + 16,), jnp.float32),
                       pltpu.VMEM((A,), jnp.float32),
                       pltpu.VMEM((A,), jnp.int32),
                       pltpu.VMEM((G,), jnp.int32),
                       pltpu.VMEM((G,), jnp.int32),
                       pltpu.VMEM((G, CPAD), jnp.int32)],
    )
    def sk(ciou_hbm, ct_hbm, cin_hbm, counts_hbm, a2g_hbm,
           civ, ctv, cinv, s1_v, mean_v, thr_v, maxiou_v, ming_v,
           cnt_v, a2g_v, ptl):
        wid = lax.axis_index("s") * 2 + lax.axis_index("c")

        @pl.when(wid < 24)
        def _():
            lvl = wid // 8
            b = wid % 8
            qbase = b * G
            for ll in range(L):
                pltpu.sync_copy(ciou_hbm.at[ll, pl.ds(qbase, G)], civ.at[ll])
            pltpu.sync_copy(ct_hbm.at[lvl, pl.ds(qbase, G)], ctv)
            pltpu.sync_copy(cin_hbm.at[lvl, pl.ds(qbase, G)], cinv)
            iota16 = lax.broadcasted_iota(jnp.int32, (16,), 0)
            masks = [(cc * 16 + iota16) < TOPK for cc in range(2)]
            lane0 = iota16 == 0

            def splat_i(s):
                return s + jnp.zeros((16,), jnp.int32)

            def sstore(ref, g, val):
                vec = val + jnp.zeros((16,), val.dtype)
                plsc.store_scatter(ref, [splat_i(g)], vec, mask=lane0)

            def sum_body(g, carry):
                acc = jnp.zeros((16,), jnp.float32)
                for ll in range(L):
                    for cc in range(2):
                        v = civ[ll, g, pl.ds(cc * 16, 16)]
                        acc = acc + jnp.where(masks[cc], v, 0.0)
                sstore(s1_v, g, jnp.sum(acc))
                return carry

            lax.fori_loop(0, G, sum_body, 0)
            for k in range(4):
                s1 = s1_v[pl.ds(k * 16, 16)]
                mean_v[pl.ds(k * 16, 16)] = s1 / 81.0

            def dev_body(g, carry):
                m = mean_v[pl.ds(g, 16)][0]
                acc = jnp.zeros((16,), jnp.float32)
                for ll in range(L):
                    for cc in range(2):
                        v = civ[ll, g, pl.ds(cc * 16, 16)] - m
                        v = jnp.where(masks[cc], v, 0.0)
                        acc = acc + v * v
                sstore(s1_v, g, jnp.sum(acc))
                return carry

            lax.fori_loop(0, G, dev_body, 0)
            for k in range(4):
                var = s1_v[pl.ds(k * 16, 16)] / 80.0
                vv = jnp.maximum(var, 1e-30)
                bi = plsc.bitcast(vv, jnp.int32)
                y = plsc.bitcast(0x5F3759DF - (bi >> 1), jnp.float32)
                for _ in range(4):
                    y = y * (1.5 - 0.5 * vv * y * y)
                std = vv * y
                thr_v[pl.ds(k * 16, 16)] = mean_v[pl.ds(k * 16, 16)] + std

            zf16 = jnp.zeros((16,), jnp.float32)
            mg16 = jnp.full((16,), MING_INIT, jnp.int32)
            for k in range(G // 16):
                maxiou_v[pl.ds(k * 16, 16)] = zf16
                ming_v[pl.ds(k * 16, 16)] = mg16

            def init_body(g, carry):
                for cc in range(2):
                    tl = ctv[g, pl.ds(cc * 16, 16)]
                    tl_c = jnp.where(masks[cc], tl, 0)
                    plsc.store_scatter(maxiou_v, [tl_c], zf16,
                                       mask=masks[cc])
                    plsc.store_scatter(ming_v, [tl_c], mg16,
                                       mask=masks[cc])
                return carry

            lax.fori_loop(0, G, init_body, 0)

            def pair_vecs(g, cc):
                iou = civ[lvl, g, pl.ds(cc * 16, 16)]
                tl = ctv[g, pl.ds(cc * 16, 16)]
                ins = cinv[g, pl.ds(cc * 16, 16)]
                thr = thr_v[pl.ds(g, 16)][0]
                pos = masks[cc] & (ins != 0) & (iou >= thr)
                tl_c = jnp.where(pos, tl, 0)
                return iou, tl_c, pos

            def smax_body(g, carry):
                for cc in range(2):
                    iou, tl_c, pos = pair_vecs(g, cc)
                    ptl[g, pl.ds(cc * 16, 16)] = jnp.where(pos, tl_c, -1)
                    cur = plsc.load_gather(maxiou_v, [tl_c])
                    upd = pos & (iou > cur)
                    plsc.store_scatter(maxiou_v, [tl_c], iou, mask=upd)
                return carry

            lax.fori_loop(0, G, smax_body, 0)

            def pair_vecs2(g, cc):
                iou = civ[lvl, g, pl.ds(cc * 16, 16)]
                tl = ptl[g, pl.ds(cc * 16, 16)]
                pos = tl >= 0
                tl_c = jnp.where(pos, tl, 0)
                return iou, tl_c, pos

            def smin_body(g, carry):
                gv = splat_i(g)
                for cc in range(2):
                    iou, tl_c, pos = pair_vecs2(g, cc)
                    mx = plsc.load_gather(maxiou_v, [tl_c])
                    el = pos & (iou == mx)
                    cur = plsc.load_gather(ming_v, [tl_c])
                    upd = el & (gv < cur)
                    plsc.store_scatter(ming_v, [tl_c], gv, mask=upd)
                return carry

            lax.fori_loop(0, G, smin_body, 0)

            def cnt_body(g, carry):
                gv = splat_i(g)
                tot = jnp.int32(0)
                for cc in range(2):
                    iou, tl_c, pos = pair_vecs2(g, cc)
                    mx = plsc.load_gather(maxiou_v, [tl_c])
                    mg = plsc.load_gather(ming_v, [tl_c])
                    win = pos & (iou == mx) & (gv == mg)
                    tot = tot + jnp.sum(jnp.where(win, 1, 0))
                sstore(cnt_v, g, tot)
                return carry

            lax.fori_loop(0, G, cnt_body, 0)
            pltpu.sync_copy(cnt_v, counts_hbm.at[b, lvl])

            @pl.when(lvl == 0)
            def _():
                for k in range(4):
                    idx = iota16 + k * 16
                    mi = plsc.load_gather(maxiou_v, [idx])
                    mg = plsc.load_gather(ming_v, [idx])
                    a2g_v[pl.ds(k * 16, 16)] = jnp.where(
                        mi > 0.0, mg, 0)
                pltpu.sync_copy(a2g_v, a2g_hbm.at[b])

    return sk(ciou, ct, cin)


def kernel(regressions, anchors, annotations):
    x0 = anchors[:, 0, :, 0]
    x1 = anchors[:, 0, :, 1]
    scx, sw, sidx = _sort_levels(x0, x1)
    g0 = annotations[:, :, 0].reshape(NQ)
    g1 = annotations[:, :, 1].reshape(NQ)
    qb = jnp.stack([g0, g1, g0 + g1, jnp.zeros_like(g0)], axis=0)
    ciou, ct, cin = _sc_query(scx, sw, sidx, qb)
    counts, a2g64 = _sc_scatter(ciou, ct, cin)
    return _finish(counts, a2g64, annotations, anchors[0, 0, :G, :],
                   regressions[:, :G, :])
